# speedup vs baseline: 1.0026x; 1.0026x over previous
import functools
import jax
import jax.numpy as jnp
from jax.experimental import pallas as pl

N = 10000
E = 320000
HID = 128
SCALE = 0.03125
BLK = 1000


def _mlp_body(s_ref, rms_w_ref, W1_ref, b1_ref, W2_ref, b2_ref, out_ref):
    x = s_ref[...]
    eps = jnp.finfo(jnp.float32).eps
    ms = jnp.mean(x * x, axis=-1, keepdims=True)
    xn = x * jax.lax.rsqrt(ms + eps) * rms_w_ref[...]
    h = jnp.dot(xn, W1_ref[...].T, preferred_element_type=jnp.float32) + b1_ref[...]
    h = h * jax.nn.sigmoid(h)
    phi = jnp.dot(h, W2_ref[...].T, preferred_element_type=jnp.float32) + b2_ref[...]
    out_ref[...] = phi


def _phi(s2d, rms_w, W1, b1, W2, b2):
    grid = (N // BLK,)
    return pl.pallas_call(
        _mlp_body,
        grid=grid,
        in_specs=[
            pl.BlockSpec((BLK, HID), lambda i: (i, 0)),
            pl.BlockSpec((HID,), lambda i: (0,)),
            pl.BlockSpec((HID, HID), lambda i: (0, 0)),
            pl.BlockSpec((HID,), lambda i: (0,)),
            pl.BlockSpec((3 * HID, HID), lambda i: (0, 0)),
            pl.BlockSpec((3 * HID,), lambda i: (0,)),
        ],
        out_specs=pl.BlockSpec((BLK, 3 * HID), lambda i: (i, 0)),
        out_shape=jax.ShapeDtypeStruct((N, 3 * HID), jnp.float32),
    )(s2d, rms_w, W1, b1, W2, b2)


def kernel(s, v, edge_index, rbf_filter, edge_vector, rms_w, W1, b1, W2, b2):
    i = edge_index[0]
    j = edge_index[1]
    phi_s = _phi(s[:, 0, :], rms_w, W1, b1, W2, b2)[:, None, :]
    filt = jnp.take(phi_s, j, axis=0) * rbf_filter
    m_s, m_vv, m_vs = jnp.split(filt, 3, axis=-1)
    ds = jax.ops.segment_sum(m_s, i, num_segments=N)
    gate = jnp.take(v, j, axis=0) * m_vv
    gate = gate + edge_vector[:, :, None] * m_vs
    dv = jax.ops.segment_sum(gate, i, num_segments=N)
    s_out = s + ds * SCALE
    v_out = v + dv * SCALE
    return (s_out, v_out)


# speedup vs baseline: 12.0266x; 11.9954x over previous
import functools
import jax
import jax.numpy as jnp
from jax import lax
from jax.experimental import pallas as pl
from jax.experimental.pallas import tpu as pltpu, tpu_sc as plsc

N = 10000
E = 320000
HID = 128
F = 3 * HID
MSG = 4 * HID
SCALE = 0.03125
BLK = 1000

NC = 2
NS = 16
NW = NC * NS
EP = E // NW
C = 16

TR = 32
NWIN = (N + TR - 1) // TR
ROUNDS = (NWIN + NS - 1) // NS
SL = 400
NSTR = EP // SL
OA_CAP = EP + 432
OA_DUMP = OA_CAP - 8
PB = 5152
PB_DUMP = 5128
THRESH = 3072
INB = 512
DUMPROW = TR
OUTR = 11000


def _mlp_body(s_ref, rms_w_ref, W1_ref, b1_ref, W2_ref, b2_ref, out_ref):
    x = s_ref[...]
    eps = jnp.finfo(jnp.float32).eps
    ms = jnp.mean(x * x, axis=-1, keepdims=True)
    xn = x * jax.lax.rsqrt(ms + eps) * rms_w_ref[...]
    h = jnp.dot(xn, W1_ref[...].T, preferred_element_type=jnp.float32) + b1_ref[...]
    h = h * jax.nn.sigmoid(h)
    phi = jnp.dot(h, W2_ref[...].T, preferred_element_type=jnp.float32) + b2_ref[...]
    out_ref[...] = phi


def _phi(s2d, rms_w, W1, b1, W2, b2):
    return pl.pallas_call(
        _mlp_body,
        grid=(N // BLK,),
        in_specs=[
            pl.BlockSpec((BLK, HID), lambda i: (i, 0)),
            pl.BlockSpec((HID,), lambda i: (0,)),
            pl.BlockSpec((HID, HID), lambda i: (0, 0)),
            pl.BlockSpec((HID,), lambda i: (0,)),
            pl.BlockSpec((F, HID), lambda i: (0, 0)),
            pl.BlockSpec((F,), lambda i: (0,)),
        ],
        out_specs=pl.BlockSpec((BLK, F), lambda i: (i, 0)),
        out_shape=jax.ShapeDtypeStruct((N, F), jnp.float32),
    )(s2d, rms_w, W1, b1, W2, b2)


def _edge_body(phi_hbm, vflat_hbm, rbf_hbm, i_hbm, j_hbm, ev_hbm, out_hbm,
               i_s, j_s, oa_e, oa_p, cnt_v, ine0, ine1, inp0, inp1, pe, pp,
               tacc, ph0, ph1, vv0, vv1, rb0, rb1, eb0, eb1,
               ij0, ij1, ie0, ie1, zbuf, ccache, xe, xp, zsh, cx,
               s00, s01, s02, s03, s10, s11, s12, s13,
               s20, s21, s22, s23):
    cid = lax.axis_index("c")
    sid = lax.axis_index("s")
    wid = cid * NS + sid
    base = wid * EP
    iota = lax.iota(jnp.int32, 16)
    i32 = jnp.int32
    slots = ((ph0, vv0, rb0, eb0, ij0, ie0, (s00, s01, s02, s03)),
             (ph1, vv1, rb1, eb1, ij1, ie1, (s10, s11, s12, s13)))
    islots = ((ine0, inp0, s20, s21), (ine1, inp1, s22, s23))

    def _zb(q, _):
        for rr8 in range(8):
            zbuf[rr8, pl.ds(q * 16, 16)] = jnp.zeros((16,), jnp.float32)
        return 0
    lax.fori_loop(0, MSG // 16, _zb, 0)

    @pl.when(sid < 5)
    def _():
        pltpu.sync_copy(zbuf, zsh.at[pl.ds(sid * 8, 8), :])

    def _cstripe(st, cnts):
        pltpu.sync_copy(i_hbm.at[pl.ds(base + st * SL, SL)], i_s)

        def _cg(g, cnts):
            iv = i_s[pl.ds(g * 16, 16)]
            o16 = jnp.bitwise_and(lax.shift_right_logical(iv, 5), 15)
            return tuple(c + jnp.sum((o16 == b).astype(i32))
                         for b, c in enumerate(cnts))

        return lax.fori_loop(0, SL // 16, _cg, cnts)

    cnts = lax.fori_loop(0, NSTR, _cstripe, (jnp.int32(0),) * NS)

    cv = jnp.zeros((16,), i32)
    for b in range(NS):
        cv = cv + jnp.where(iota == b, cnts[b], 0)
    p8 = jnp.bitwise_and(cv + 7, -8)
    cs = plsc.cumsum(p8)
    astart_v = cs - p8
    cnt_v[...] = cv
    pltpu.sync_copy(cnt_v, cx.at[pl.ds(sid * 16, 16)])
    astart_sc = [astart_v[b] for b in range(NS)]

    def _pstripe(st, runs):
        so = st * SL
        pltpu.sync_copy(i_hbm.at[pl.ds(base + so, SL)], i_s)
        pltpu.sync_copy(j_hbm.at[pl.ds(base + so, SL)], j_s)

        def _pg(g, runs):
            iv = i_s[pl.ds(g * 16, 16)]
            jv = j_s[pl.ds(g * 16, 16)]
            e16 = base + so + g * 16 + iota
            o16 = jnp.bitwise_and(lax.shift_right_logical(iv, 5), 15)
            pv16 = (jnp.left_shift(jv, 10)
                    | jnp.left_shift(jnp.bitwise_and(iv, 31), 5)
                    | lax.shift_right_logical(iv, 9))
            new_runs = []
            for b in range(NS):
                m = o16 == b
                cum = plsc.cumsum(m.astype(i32))
                posb = jnp.where(m, astart_sc[b] + runs[b] + cum - 1, OA_DUMP)
                plsc.store_scatter(oa_e, [posb], e16)
                plsc.store_scatter(oa_p, [posb], pv16)
                new_runs.append(runs[b] + cum[15])
            return tuple(new_runs)

        return lax.fori_loop(0, SL // 16, _pg, runs)

    lax.fori_loop(0, NSTR, _pstripe, (jnp.int32(0),) * NS)

    pltpu.sync_copy(oa_e, xe.at[pl.ds(sid * OA_CAP, OA_CAP)])
    pltpu.sync_copy(oa_p, xp.at[pl.ds(sid * OA_CAP, OA_CAP)])
    plsc.subcore_barrier()
    pltpu.sync_copy(cx, ccache)

    def _issue(n, slot, vc):
        ph, vv, rb, eb, ij, ie, sems = slots[slot]
        ok = iota < vc
        e16 = jnp.where(ok, pe[pl.ds(n * 16, 16)], 0)
        pv16 = jnp.where(ok, pp[pl.ds(n * 16, 16)], 0)
        ij[...] = lax.shift_right_logical(pv16, 10)
        ie[...] = e16
        pltpu.async_copy(phi_hbm.at[ij], ph, sems[0])
        pltpu.async_copy(vflat_hbm.at[ij], vv, sems[1])
        pltpu.async_copy(rbf_hbm.at[ie], rb, sems[2])
        pltpu.async_copy(ev_hbm.at[ie], eb, sems[3])

    def _wait(slot):
        ph, vv, rb, eb, ij, ie, sems = slots[slot]
        pltpu.make_async_copy(phi_hbm.at[ij], ph, sems[0]).wait()
        pltpu.make_async_copy(vflat_hbm.at[ij], vv, sems[1]).wait()
        pltpu.make_async_copy(rbf_hbm.at[ie], rb, sems[2]).wait()
        pltpu.make_async_copy(ev_hbm.at[ie], eb, sems[3]).wait()

    def _compute(n, slot, vc):
        ph, vv, rb, eb, ij, ie, sems = slots[slot]

        def _edge(ee, _):
            pvv = plsc.load_gather(pp, [jnp.full((16,), n * 16 + ee, i32)])
            il = jnp.where(ee < vc,
                           jnp.bitwise_and(lax.shift_right_logical(pvv[0], 5), 31),
                           DUMPROW)
            ve = plsc.load_gather(eb, [jnp.full((16,), ee, i32), iota])
            ilv = jnp.full((16,), il, i32)
            for q in range(8):
                fs = (ph[ee, pl.ds(q * 16, 16)] * rb[ee, pl.ds(q * 16, 16)])
                plsc.addupdate_scatter(tacc, [ilv, q * 16 + iota], fs)
            mvv = []
            mvs = []
            for q in range(8):
                mvv.append(ph[ee, pl.ds(HID + q * 16, 16)]
                           * rb[ee, pl.ds(HID + q * 16, 16)])
                mvs.append(ph[ee, pl.ds(2 * HID + q * 16, 16)]
                           * rb[ee, pl.ds(2 * HID + q * 16, 16)])
            for k in range(3):
                evk = ve[k]
                for q in range(8):
                    g = (vv[ee, pl.ds(k * HID + q * 16, 16)] * mvv[q]
                         + evk * mvs[q])
                    plsc.addupdate_scatter(tacc, [ilv, HID + k * HID + q * 16 + iota], g)
            return 0

        lax.fori_loop(0, 16, _edge, 0)

    full16 = jnp.int32(16)

    def _drain(pcnt):
        nch = pcnt // 16

        @pl.when(nch > 0)
        def _():
            _issue(0, 0, full16)

        def _pair(p, _):
            n0 = 2 * p
            n1 = n0 + 1

            @pl.when(n1 < nch)
            def _():
                _issue(n1, 1, full16)

            _wait(0)
            _compute(n0, 0, full16)

            @pl.when(n1 + 1 < nch)
            def _():
                _issue(n1 + 1, 0, full16)

            @pl.when(n1 < nch)
            def _():
                _wait(1)
                _compute(n1, 1, full16)

            return 0

        lax.fori_loop(0, (nch + 1) // 2, _pair, 0)
        rem = pcnt - nch * 16

        @pl.when(nch > 0)
        def _():
            re16 = pe[pl.ds(nch * 16, 16)]
            rp16 = pp[pl.ds(nch * 16, 16)]
            pe[pl.ds(0, 16)] = re16
            pp[pl.ds(0, 16)] = rp16

        return rem

    def _round(rr, _):
        pltpu.sync_copy(zsh, tacc)

        def _src(t, pcnt):
            cvec = ccache[pl.ds(t * 16, 16)]
            p8t = jnp.bitwise_and(cvec + 7, -8)
            cst = plsc.cumsum(p8t)
            astart_t = cst - p8t
            lmine = jnp.sum(jnp.where(iota == sid, cvec, 0))
            amine = pl.multiple_of(jnp.sum(jnp.where(iota == sid, astart_t, 0)), 8)
            nin = (lmine + INB - 1) // INB
            ibase = t * OA_CAP + amine

            def _issue_in(c, slot):
                sl_e, sl_p, se, sp = islots[slot]
                pltpu.async_copy(xe.at[pl.ds(ibase + c * INB, INB)], sl_e, se)
                pltpu.async_copy(xp.at[pl.ds(ibase + c * INB, INB)], sl_p, sp)

            def _wait_in(c, slot):
                sl_e, sl_p, se, sp = islots[slot]
                pltpu.make_async_copy(xe.at[pl.ds(ibase + c * INB, INB)], sl_e, se).wait()
                pltpu.make_async_copy(xp.at[pl.ds(ibase + c * INB, INB)], sl_p, sp).wait()

            def _filter(c, slot, pcnt):
                sl_e, sl_p, _se, _sp = islots[slot]

                def _g(g, pcnt):
                    ee16 = sl_e[pl.ds(g * 16, 16)]
                    pv16 = sl_p[pl.ds(g * 16, 16)]
                    ok = jnp.logical_and(c * INB + g * 16 + iota < lmine,
                                         jnp.bitwise_and(pv16, 31) == rr)
                    cum = plsc.cumsum(ok.astype(i32))
                    posn = jnp.where(ok, pcnt + cum - 1, PB_DUMP)
                    plsc.store_scatter(pe, [posn], ee16)
                    plsc.store_scatter(pp, [posn], pv16)
                    return pcnt + cum[15]

                pcnt = lax.fori_loop(0, INB // 16, _g, pcnt)
                return lax.cond(pcnt >= THRESH, _drain, lambda x: x, pcnt)

            @pl.when(nin > 0)
            def _():
                _issue_in(0, 0)

            def _pairin(p, pcnt):
                c0 = 2 * p
                c1 = c0 + 1

                @pl.when(c1 < nin)
                def _():
                    _issue_in(c1, 1)

                _wait_in(c0, 0)
                pcnt = _filter(c0, 0, pcnt)

                @pl.when(c1 + 1 < nin)
                def _():
                    _issue_in(c1 + 1, 0)

                def _do1(pc):
                    _wait_in(c1, 1)
                    return _filter(c1, 1, pc)

                return lax.cond(c1 < nin, _do1, lambda pc: pc, pcnt)

            return lax.fori_loop(0, (nin + 1) // 2, _pairin, pcnt)

        pcnt = lax.fori_loop(0, NS, _src, jnp.int32(0))
        pcnt = _drain(pcnt)

        @pl.when(pcnt > 0)
        def _():
            _issue(0, 0, pcnt)
            _wait(0)
            _compute(0, 0, pcnt)

        w = rr * NS + sid

        @pl.when(w < NWIN)
        def _():
            pltpu.sync_copy(tacc.at[pl.ds(0, TR), :],
                            out_hbm.at[pl.ds(cid * OUTR + w * TR, TR), :])

        return 0

    lax.fori_loop(0, ROUNDS, _round, 0)


def _edge_messages(phi, vflat, rbf, i_arr, j_arr, ev4):
    mesh = plsc.VectorSubcoreMesh(core_axis_name="c", subcore_axis_name="s")
    f32 = jnp.float32
    i32 = jnp.int32
    fn = pl.kernel(
        _edge_body,
        out_type=jax.ShapeDtypeStruct((2 * OUTR, MSG), f32),
        mesh=mesh,
        compiler_params=pltpu.CompilerParams(needs_layout_passes=False, use_tc_tiling_on_sc=False),
        scratch_types=[
            pltpu.VMEM((SL,), i32),
            pltpu.VMEM((SL,), i32),
            pltpu.VMEM((OA_CAP,), i32),
            pltpu.VMEM((OA_CAP,), i32),
            pltpu.VMEM((16,), i32),
            pltpu.VMEM((INB,), i32),
            pltpu.VMEM((INB,), i32),
            pltpu.VMEM((INB,), i32),
            pltpu.VMEM((INB,), i32),
            pltpu.VMEM((PB,), i32),
            pltpu.VMEM((PB,), i32),
            pltpu.VMEM((TR + 8, MSG), f32),
            pltpu.VMEM((C, F), f32),
            pltpu.VMEM((C, F), f32),
            pltpu.VMEM((C, F), f32),
            pltpu.VMEM((C, F), f32),
            pltpu.VMEM((C, F), f32),
            pltpu.VMEM((C, F), f32),
            pltpu.VMEM((C, HID), f32),
            pltpu.VMEM((C, HID), f32),
            pltpu.VMEM((C,), i32),
            pltpu.VMEM((C,), i32),
            pltpu.VMEM((C,), i32),
            pltpu.VMEM((C,), i32),
            pltpu.VMEM((8, MSG), f32),
            pltpu.VMEM((NS * 16,), i32),
            pltpu.VMEM_SHARED((NS * OA_CAP + INB,), i32),
            pltpu.VMEM_SHARED((NS * OA_CAP + INB,), i32),
            pltpu.VMEM_SHARED((TR + 8, MSG), f32),
            pltpu.VMEM_SHARED((NS * 16,), i32),
            pltpu.SemaphoreType.DMA,
            pltpu.SemaphoreType.DMA,
            pltpu.SemaphoreType.DMA,
            pltpu.SemaphoreType.DMA,
            pltpu.SemaphoreType.DMA,
            pltpu.SemaphoreType.DMA,
            pltpu.SemaphoreType.DMA,
            pltpu.SemaphoreType.DMA,
            pltpu.SemaphoreType.DMA,
            pltpu.SemaphoreType.DMA,
            pltpu.SemaphoreType.DMA,
            pltpu.SemaphoreType.DMA,
        ],
    )
    return fn(phi, vflat, rbf, i_arr, j_arr, ev4)


def _combine_body(s_ref, v_ref, p0_ref, p1_ref, so_ref, vo_ref):
    tot = p0_ref[...] + p1_ref[...]
    so_ref[...] = s_ref[...] + SCALE * tot[:, :HID]
    vo_ref[...] = v_ref[...] + SCALE * tot[:, HID:]


def _combine(s2d, v2d, partial):
    nb = N // BLK
    return pl.pallas_call(
        _combine_body,
        grid=(nb,),
        in_specs=[
            pl.BlockSpec((BLK, HID), lambda i: (i, 0)),
            pl.BlockSpec((BLK, F), lambda i: (i, 0)),
            pl.BlockSpec((BLK, MSG), lambda i: (i, 0)),
            pl.BlockSpec((BLK, MSG), lambda i: (i + OUTR // BLK, 0)),
        ],
        out_specs=[
            pl.BlockSpec((BLK, HID), lambda i: (i, 0)),
            pl.BlockSpec((BLK, F), lambda i: (i, 0)),
        ],
        out_shape=[
            jax.ShapeDtypeStruct((N, HID), jnp.float32),
            jax.ShapeDtypeStruct((N, F), jnp.float32),
        ],
    )(s2d, v2d, partial, partial)


def kernel(s, v, edge_index, rbf_filter, edge_vector, rms_w, W1, b1, W2, b2):
    s2d = s[:, 0, :]
    v2d = v.reshape(N, F)
    phi = _phi(s2d, rms_w, W1, b1, W2, b2)
    ev4 = jnp.pad(edge_vector, ((0, 0), (0, 125)))
    partial = _edge_messages(phi, v2d, rbf_filter[:, 0, :],
                             edge_index[0], edge_index[1], ev4)
    s_out, v_out = _combine(s2d, v2d, partial)
    return (s_out[:, None, :], v_out.reshape(N, 3, HID))
